# baseline (speedup 1.0000x reference)
import jax
import jax.numpy as jnp
from jax import lax
from jax.experimental import pallas as pl
from jax.experimental.pallas import tpu as pltpu
from jax.experimental.pallas import tpu_sc as plsc

NUM_CONTEXTS = 100000
Z_DIM = 128
BATCH = 16384

_info = plsc.get_sparse_core_info()
_NC, _NS = _info.num_cores, _info.num_subcores
_NW = _NC * _NS
_B_PER_W = BATCH // _NW
_CHUNK = 128
_NCHUNK = _B_PER_W // _CHUNK


def _gather_body(labels_hbm, table_hbm, out_hbm, idx_v, rows_v,
                 gsems, ssem):
    wid = lax.axis_index("s") * _NC + lax.axis_index("c")
    base = wid * _B_PER_W
    for j in range(_NCHUNK):
        pltpu.sync_copy(labels_hbm.at[pl.ds(base + j * _CHUNK, _CHUNK)],
                        idx_v.at[j])
    gathers = []
    for j in range(_NCHUNK):
        gathers.append(pltpu.async_copy(
            table_hbm.at[idx_v.at[j]],
            rows_v.at[pl.ds(j * _CHUNK, _CHUNK)],
            gsems[j]))
    stores = []
    for j in range(_NCHUNK):
        gathers[j].wait()
        stores.append(pltpu.async_copy(
            rows_v.at[pl.ds(j * _CHUNK, _CHUNK)],
            out_hbm.at[pl.ds(base + j * _CHUNK, _CHUNK)],
            ssem))
    for s in stores:
        s.wait()


@jax.jit
def _embed_gather(context_labels, embed_table):
    mesh = plsc.VectorSubcoreMesh(core_axis_name="c", subcore_axis_name="s")
    return pl.kernel(
        _gather_body,
        out_type=jax.ShapeDtypeStruct((BATCH, Z_DIM), jnp.float32),
        mesh=mesh,
        scratch_types=[
            pltpu.VMEM((_NCHUNK, _CHUNK), jnp.int32),
            pltpu.VMEM((_B_PER_W, Z_DIM), jnp.float32),
            [pltpu.SemaphoreType.DMA] * _NCHUNK,
            pltpu.SemaphoreType.DMA,
        ],
    )(context_labels, embed_table)


def kernel(context_labels, embed_table):
    return _embed_gather(context_labels.astype(jnp.int32), embed_table)


# speedup vs baseline: 1.0489x; 1.0489x over previous
import jax
import jax.numpy as jnp
from jax import lax
from jax.experimental import pallas as pl
from jax.experimental.pallas import tpu as pltpu
from jax.experimental.pallas import tpu_sc as plsc

NUM_CONTEXTS = 100000
Z_DIM = 128
BATCH = 16384

_info = plsc.get_sparse_core_info()
_NC, _NS = _info.num_cores, _info.num_subcores
_NW = _NC * _NS
_B_PER_W = BATCH // _NW
_CHUNK = 128
_NCHUNK = _B_PER_W // _CHUNK


def _gather_body(labels_hbm, table_hbm, out_hbm, idx_v, rows_v, gsems, ssem):
    wid = lax.axis_index("s") * _NC + lax.axis_index("c")
    base = wid * _B_PER_W
    pltpu.sync_copy(labels_hbm.at[wid], idx_v)
    gathers = []
    for j in range(_NCHUNK):
        gathers.append(pltpu.async_copy(
            table_hbm.at[idx_v.at[j]],
            rows_v.at[pl.ds(j * _CHUNK, _CHUNK)],
            gsems[j]))
    stores = []
    for j in range(_NCHUNK):
        gathers[j].wait()
        stores.append(pltpu.async_copy(
            rows_v.at[pl.ds(j * _CHUNK, _CHUNK)],
            out_hbm.at[pl.ds(base + j * _CHUNK, _CHUNK)],
            ssem))
    for s in stores:
        s.wait()


@jax.jit
def _embed_gather(labels_3d, embed_table):
    mesh = plsc.VectorSubcoreMesh(core_axis_name="c", subcore_axis_name="s")
    return pl.kernel(
        _gather_body,
        out_type=jax.ShapeDtypeStruct((BATCH, Z_DIM), jnp.float32),
        mesh=mesh,
        scratch_types=[
            pltpu.VMEM((_NCHUNK, _CHUNK), jnp.int32),
            pltpu.VMEM((_B_PER_W, Z_DIM), jnp.float32),
            [pltpu.SemaphoreType.DMA] * _NCHUNK,
            pltpu.SemaphoreType.DMA,
        ],
    )(labels_3d, embed_table)


def kernel(context_labels, embed_table):
    labels_3d = context_labels.astype(jnp.int32).reshape(_NW, _NCHUNK, _CHUNK)
    return _embed_gather(labels_3d, embed_table)


# speedup vs baseline: 1.0532x; 1.0041x over previous
import jax
import jax.numpy as jnp
from jax import lax
from jax.experimental import pallas as pl
from jax.experimental.pallas import tpu as pltpu
from jax.experimental.pallas import tpu_sc as plsc

NUM_CONTEXTS = 100000
Z_DIM = 128
BATCH = 16384

_info = plsc.get_sparse_core_info()
_NC, _NS = _info.num_cores, _info.num_subcores
_NW = _NC * _NS
_B_PER_W = BATCH // _NW


def _gather_body(labels_hbm, table_hbm, out_hbm, idx_v, rows_v, sem):
    wid = lax.axis_index("s") * _NC + lax.axis_index("c")
    base = wid * _B_PER_W
    pltpu.sync_copy(labels_hbm.at[wid], idx_v)
    pltpu.async_copy(table_hbm.at[idx_v], rows_v, sem).wait()
    pltpu.sync_copy(rows_v, out_hbm.at[pl.ds(base, _B_PER_W)])


@jax.jit
def _embed_gather(labels_2d, embed_table):
    mesh = plsc.VectorSubcoreMesh(core_axis_name="c", subcore_axis_name="s")
    return pl.kernel(
        _gather_body,
        out_type=jax.ShapeDtypeStruct((BATCH, Z_DIM), jnp.float32),
        mesh=mesh,
        scratch_types=[
            pltpu.VMEM((_B_PER_W,), jnp.int32),
            pltpu.VMEM((_B_PER_W, Z_DIM), jnp.float32),
            pltpu.SemaphoreType.DMA,
        ],
    )(labels_2d, embed_table)


def kernel(context_labels, embed_table):
    labels_2d = context_labels.astype(jnp.int32).reshape(_NW, _B_PER_W)
    return _embed_gather(labels_2d, embed_table)


# speedup vs baseline: 1.0670x; 1.0131x over previous
import jax
import jax.numpy as jnp
from jax import lax
from jax.experimental import pallas as pl
from jax.experimental.pallas import tpu as pltpu
from jax.experimental.pallas import tpu_sc as plsc

NUM_CONTEXTS = 100000
Z_DIM = 128
BATCH = 16384

_info = plsc.get_sparse_core_info()
_NC, _NS = _info.num_cores, _info.num_subcores
_NW = _NC * _NS
_B_PER_W = BATCH // _NW


def _gather_body(labels_hbm, table_hbm, out_hbm, idx_v, rows_v, sem):
    wid = lax.axis_index("s") * _NC + lax.axis_index("c")
    base = wid * _B_PER_W
    pltpu.sync_copy(labels_hbm.at[pl.ds(base, _B_PER_W)], idx_v)
    pltpu.async_copy(table_hbm.at[idx_v], rows_v, sem).wait()
    pltpu.sync_copy(rows_v, out_hbm.at[pl.ds(base, _B_PER_W)])


@jax.jit
def _embed_gather(context_labels, embed_table):
    mesh = plsc.VectorSubcoreMesh(core_axis_name="c", subcore_axis_name="s")
    return pl.kernel(
        _gather_body,
        out_type=jax.ShapeDtypeStruct((BATCH, Z_DIM), jnp.float32),
        mesh=mesh,
        scratch_types=[
            pltpu.VMEM((_B_PER_W,), jnp.int32),
            pltpu.VMEM((_B_PER_W, Z_DIM), jnp.float32),
            pltpu.SemaphoreType.DMA,
        ],
    )(context_labels, embed_table)


def kernel(context_labels, embed_table):
    return _embed_gather(context_labels.astype(jnp.int32), embed_table)
